# Initial kernel scaffold; baseline (speedup 1.0000x reference)
import jax
import jax.numpy as jnp
from jax.experimental import pallas as pl


def kernel(X, A, W1, b1, W2, b2, W3, b3, Wfc, bfc):
    raise NotImplementedError("write your pallas kernel here")


# speedup vs baseline: 2.7022x; 2.7022x over previous
import jax
import jax.numpy as jnp
from jax.experimental import pallas as pl

B, N, T, C_IN = 8, 1000, 12, 64
OUT_CH = 4
ATT_DIM = T - 2
HALF = ATT_DIM * OUT_CH
I_BLK = 128
N_IBLK = (N + I_BLK - 1) // I_BLK


def _score_kernel(x_ref, w_ref, b_ref, wa_ref, wb_ref, s1_ref, s2_ref):
    x = x_ref[0]
    outs = []
    for j in range(3):
        acc = jnp.zeros((N * ATT_DIM, OUT_CH), dtype=jnp.float32)
        for k in range(3):
            xs = x[:, k:k + ATT_DIM, :].reshape(N * ATT_DIM, C_IN)
            acc = acc + jnp.dot(xs, w_ref[j, k],
                                preferred_element_type=jnp.float32)
        outs.append(acc + b_ref[j][None, :])
    o1, o2, o3 = outs
    t = jax.nn.relu(o1 + jax.nn.sigmoid(o2) + o3)
    t3 = t.reshape(N, ATT_DIM, OUT_CH)
    s1_ref[0, 0, :] = jnp.sum(t3 * wa_ref[...][None, :, :], axis=(1, 2))
    s2_ref[0, 0, :] = jnp.sum(t3 * wb_ref[...][None, :, :], axis=(1, 2))


def _attn_kernel(s1_ref, s2_ref, a_ref, bfc_ref, out_ref):
    raw = s1_ref[0, 0, :][:, None] + s2_ref[0, 0, :][None, :] + bfc_ref[0, 0]
    scores = jnp.where(raw >= 0, raw, 0.01 * raw)
    a = a_ref[...]
    val = jnp.where(a != 0, scores, 0.0)
    m = jnp.max(val, axis=1, keepdims=True)
    e = jnp.exp(val - m)
    s = jnp.sum(e, axis=1, keepdims=True)
    out_ref[0] = e / s * a


@jax.jit
def kernel(X, A, W1, b1, W2, b2, W3, b3, Wfc, bfc):
    W = jnp.stack([jnp.transpose(w[:, :, 0, :], (2, 1, 0))
                   for w in (W1, W2, W3)])
    bvec = jnp.stack([b1, b2, b3])
    wa = Wfc[0, :HALF].reshape(ATT_DIM, OUT_CH)
    wb = Wfc[0, HALF:].reshape(ATT_DIM, OUT_CH)
    bfc2 = bfc.reshape(1, 1)

    s1, s2 = pl.pallas_call(
        _score_kernel,
        grid=(B,),
        in_specs=[
            pl.BlockSpec((1, N, T, C_IN), lambda b: (b, 0, 0, 0)),
            pl.BlockSpec((3, 3, C_IN, OUT_CH), lambda b: (0, 0, 0, 0)),
            pl.BlockSpec((3, OUT_CH), lambda b: (0, 0)),
            pl.BlockSpec((ATT_DIM, OUT_CH), lambda b: (0, 0)),
            pl.BlockSpec((ATT_DIM, OUT_CH), lambda b: (0, 0)),
        ],
        out_specs=[
            pl.BlockSpec((1, 1, N), lambda b: (b, 0, 0)),
            pl.BlockSpec((1, 1, N), lambda b: (b, 0, 0)),
        ],
        out_shape=[
            jax.ShapeDtypeStruct((B, 1, N), jnp.float32),
            jax.ShapeDtypeStruct((B, 1, N), jnp.float32),
        ],
    )(X, W, bvec, wa, wb)

    return pl.pallas_call(
        _attn_kernel,
        grid=(B, N_IBLK),
        in_specs=[
            pl.BlockSpec((1, 1, I_BLK), lambda b, i: (b, 0, i)),
            pl.BlockSpec((1, 1, N), lambda b, i: (b, 0, 0)),
            pl.BlockSpec((I_BLK, N), lambda b, i: (i, 0)),
            pl.BlockSpec((1, 1), lambda b, i: (0, 0)),
        ],
        out_specs=pl.BlockSpec((1, I_BLK, N), lambda b, i: (b, i, 0)),
        out_shape=jax.ShapeDtypeStruct((B, N, N), jnp.float32),
    )(s1, s2, A, bfc2)


# speedup vs baseline: 7.8076x; 2.8894x over previous
import numpy as np

import jax
import jax.numpy as jnp
from jax.experimental import pallas as pl

B, N, T, C_IN = 8, 1000, 12, 64
OUT_CH = 4
ATT_DIM = T - 2
HALF = ATT_DIM * OUT_CH
LANE = 128
I_BLK = 128
N_IBLK = (N + I_BLK - 1) // I_BLK

_SEL = np.zeros((3, T, ATT_DIM), dtype=np.float32)
for _k in range(3):
    for _t in range(ATT_DIM):
        _SEL[_k, _t + _k, _t] = 1.0


def _score_kernel(x_ref, m_ref, brow_ref, wab_ref, s_ref):
    x = x_ref[0]
    y = jnp.dot(x, m_ref[...], preferred_element_type=jnp.float32)
    o1 = y[:, 0:LANE] + brow_ref[0, :][None, :]
    o2 = y[:, LANE:2 * LANE] + brow_ref[1, :][None, :]
    o3 = y[:, 2 * LANE:3 * LANE] + brow_ref[2, :][None, :]
    t = jax.nn.relu(o1 + jax.nn.sigmoid(o2) + o3)
    s_ref[0] = jax.lax.dot_general(
        wab_ref[...], t, (((1,), (1,)), ((), ())),
        preferred_element_type=jnp.float32)


def _attn_kernel(s1_ref, s2_ref, a_ref, bfc_ref, out_ref):
    raw = s1_ref[0, 0, :][:, None] + s2_ref[0, 1, :][None, :] + bfc_ref[0, 0]
    scores = jnp.where(raw >= 0, raw, 0.01 * raw)
    a = a_ref[...]
    val = jnp.where(a != 0, scores, 0.0)
    m = jnp.max(val, axis=1, keepdims=True)
    e = jnp.exp(val - m)
    s = jnp.sum(e, axis=1, keepdims=True)
    out_ref[0] = e * (1.0 / s) * a


@jax.jit
def kernel(X, A, W1, b1, W2, b2, W3, b3, Wfc, bfc):
    W = jnp.stack([jnp.transpose(w[:, :, 0, :], (2, 1, 0))
                   for w in (W1, W2, W3)])
    sel = jnp.asarray(_SEL)
    M0 = jnp.einsum('kpt,jkco->pcjto', sel, W)
    M1 = M0.reshape(T * C_IN, 3, HALF)
    M = jnp.pad(M1, ((0, 0), (0, 0), (0, LANE - HALF))).reshape(
        T * C_IN, 3 * LANE)
    brow = jnp.pad(
        jnp.tile(jnp.stack([b1, b2, b3])[:, None, :], (1, ATT_DIM, 1))
        .reshape(3, HALF), ((0, 0), (0, LANE - HALF)))
    wab = jnp.pad(Wfc.reshape(2, HALF), ((0, 0), (0, LANE - HALF)))
    bfc2 = bfc.reshape(1, 1)
    Xf = X.reshape(B, N, T * C_IN)

    s_all = pl.pallas_call(
        _score_kernel,
        grid=(B,),
        in_specs=[
            pl.BlockSpec((1, N, T * C_IN), lambda b: (b, 0, 0)),
            pl.BlockSpec((T * C_IN, 3 * LANE), lambda b: (0, 0)),
            pl.BlockSpec((3, LANE), lambda b: (0, 0)),
            pl.BlockSpec((2, LANE), lambda b: (0, 0)),
        ],
        out_specs=pl.BlockSpec((1, 2, N), lambda b: (b, 0, 0)),
        out_shape=jax.ShapeDtypeStruct((B, 2, N), jnp.float32),
    )(Xf, M, brow, wab)

    return pl.pallas_call(
        _attn_kernel,
        grid=(N_IBLK, B),
        in_specs=[
            pl.BlockSpec((1, 2, I_BLK), lambda i, b: (b, 0, i)),
            pl.BlockSpec((1, 2, N), lambda i, b: (b, 0, 0)),
            pl.BlockSpec((I_BLK, N), lambda i, b: (i, 0)),
            pl.BlockSpec((1, 1), lambda i, b: (0, 0)),
        ],
        out_specs=pl.BlockSpec((1, I_BLK, N), lambda i, b: (b, i, 0)),
        out_shape=jax.ShapeDtypeStruct((B, N, N), jnp.float32),
    )(s_all, s_all, A, bfc2)


# speedup vs baseline: 9.3337x; 1.1955x over previous
import numpy as np

import jax
import jax.numpy as jnp
from jax.experimental import pallas as pl

B, N, T, C_IN = 8, 1000, 12, 64
OUT_CH = 4
ATT_DIM = T - 2
HALF = ATT_DIM * OUT_CH
LANE = 128
I_BLK = 256
N_IBLK = (N + I_BLK - 1) // I_BLK

_SEL = np.zeros((3, T, ATT_DIM), dtype=np.float32)
for _k in range(3):
    for _t in range(ATT_DIM):
        _SEL[_k, _t + _k, _t] = 1.0


def _score_kernel(x_ref, m_ref, brow_ref, wab_ref, s_ref):
    x = x_ref[0]
    y = jnp.dot(x, m_ref[...], preferred_element_type=jnp.float32)
    o1 = y[:, 0:LANE] + brow_ref[0, :][None, :]
    o2 = y[:, LANE:2 * LANE] + brow_ref[1, :][None, :]
    o3 = y[:, 2 * LANE:3 * LANE] + brow_ref[2, :][None, :]
    t = jax.nn.relu(o1 + jax.nn.sigmoid(o2) + o3)
    s_ref[0] = jax.lax.dot_general(
        wab_ref[...], t, (((1,), (1,)), ((), ())),
        preferred_element_type=jnp.float32)


def _attn_kernel(s1_ref, s2_ref, a_ref, bfc_ref, out_ref):
    raw = s1_ref[0, 0, :][:, None] + s2_ref[0, 1, :][None, :] + bfc_ref[0, 0]
    scores = jnp.where(raw >= 0, raw, 0.01 * raw)
    a = a_ref[...]
    val = jnp.where(a != 0, scores, 0.0)
    m = jnp.max(val, axis=1, keepdims=True)
    e = jnp.exp(val - m)
    s = jnp.sum(e, axis=1, keepdims=True)
    out_ref[0] = e * (1.0 / s) * a


@jax.jit
def kernel(X, A, W1, b1, W2, b2, W3, b3, Wfc, bfc):
    W = jnp.stack([jnp.transpose(w[:, :, 0, :], (2, 1, 0))
                   for w in (W1, W2, W3)])
    sel = jnp.asarray(_SEL)
    M0 = jnp.einsum('kpt,jkco->pcjto', sel, W)
    M1 = M0.reshape(T * C_IN, 3, HALF)
    M = jnp.pad(M1, ((0, 0), (0, 0), (0, LANE - HALF))).reshape(
        T * C_IN, 3 * LANE)
    brow = jnp.pad(
        jnp.tile(jnp.stack([b1, b2, b3])[:, None, :], (1, ATT_DIM, 1))
        .reshape(3, HALF), ((0, 0), (0, LANE - HALF)))
    wab = jnp.pad(Wfc.reshape(2, HALF), ((0, 0), (0, LANE - HALF)))
    bfc2 = bfc.reshape(1, 1)
    Xf = X.reshape(B, N, T * C_IN)

    s_all = pl.pallas_call(
        _score_kernel,
        grid=(B,),
        in_specs=[
            pl.BlockSpec((1, N, T * C_IN), lambda b: (b, 0, 0)),
            pl.BlockSpec((T * C_IN, 3 * LANE), lambda b: (0, 0)),
            pl.BlockSpec((3, LANE), lambda b: (0, 0)),
            pl.BlockSpec((2, LANE), lambda b: (0, 0)),
        ],
        out_specs=pl.BlockSpec((1, 2, N), lambda b: (b, 0, 0)),
        out_shape=jax.ShapeDtypeStruct((B, 2, N), jnp.float32),
    )(Xf, M, brow, wab)

    return pl.pallas_call(
        _attn_kernel,
        grid=(N_IBLK, B),
        in_specs=[
            pl.BlockSpec((1, 2, I_BLK), lambda i, b: (b, 0, i)),
            pl.BlockSpec((1, 2, N), lambda i, b: (b, 0, 0)),
            pl.BlockSpec((I_BLK, N), lambda i, b: (i, 0)),
            pl.BlockSpec((1, 1), lambda i, b: (0, 0)),
        ],
        out_specs=pl.BlockSpec((1, I_BLK, N), lambda i, b: (b, i, 0)),
        out_shape=jax.ShapeDtypeStruct((B, N, N), jnp.float32),
    )(s_all, s_all, A, bfc2)


# speedup vs baseline: 10.4565x; 1.1203x over previous
import numpy as np

import jax
import jax.numpy as jnp
from jax.experimental import pallas as pl

B, N, T, C_IN = 8, 1000, 12, 64
OUT_CH = 4
ATT_DIM = T - 2
HALF = ATT_DIM * OUT_CH
LANE = 128
I_BLK = 512
N_IBLK = (N + I_BLK - 1) // I_BLK

_SEL = np.zeros((3, T, ATT_DIM), dtype=np.float32)
for _k in range(3):
    for _t in range(ATT_DIM):
        _SEL[_k, _t + _k, _t] = 1.0


def _score_kernel(x_ref, m_ref, brow_ref, wab_ref, s_ref):
    x = x_ref[0]
    y = jnp.dot(x, m_ref[...], preferred_element_type=jnp.float32)
    o1 = y[:, 0:LANE] + brow_ref[0, :][None, :]
    o2 = y[:, LANE:2 * LANE] + brow_ref[1, :][None, :]
    o3 = y[:, 2 * LANE:3 * LANE] + brow_ref[2, :][None, :]
    t = jax.nn.relu(o1 + jax.nn.sigmoid(o2) + o3)
    s_ref[0] = jax.lax.dot_general(
        wab_ref[...], t, (((1,), (1,)), ((), ())),
        preferred_element_type=jnp.float32)


def _attn_kernel(s1_ref, s2_ref, a_ref, bfc_ref, out_ref):
    raw = s1_ref[0, 0, :][:, None] + s2_ref[0, 1, :][None, :] + bfc_ref[0, 0]
    scores = jnp.where(raw >= 0, raw, 0.01 * raw)
    a = a_ref[...]
    val = jnp.where(a != 0, scores, 0.0)
    m = jnp.max(val, axis=1, keepdims=True)
    e = jnp.exp(val - m)
    s = jnp.sum(e, axis=1, keepdims=True)
    out_ref[0] = e * (1.0 / s) * a


@jax.jit
def kernel(X, A, W1, b1, W2, b2, W3, b3, Wfc, bfc):
    W = jnp.stack([jnp.transpose(w[:, :, 0, :], (2, 1, 0))
                   for w in (W1, W2, W3)])
    sel = jnp.asarray(_SEL)
    M0 = jnp.einsum('kpt,jkco->pcjto', sel, W)
    M1 = M0.reshape(T * C_IN, 3, HALF)
    M = jnp.pad(M1, ((0, 0), (0, 0), (0, LANE - HALF))).reshape(
        T * C_IN, 3 * LANE)
    brow = jnp.pad(
        jnp.tile(jnp.stack([b1, b2, b3])[:, None, :], (1, ATT_DIM, 1))
        .reshape(3, HALF), ((0, 0), (0, LANE - HALF)))
    wab = jnp.pad(Wfc.reshape(2, HALF), ((0, 0), (0, LANE - HALF)))
    bfc2 = bfc.reshape(1, 1)
    Xf = X.reshape(B, N, T * C_IN)

    s_all = pl.pallas_call(
        _score_kernel,
        grid=(B,),
        in_specs=[
            pl.BlockSpec((1, N, T * C_IN), lambda b: (b, 0, 0)),
            pl.BlockSpec((T * C_IN, 3 * LANE), lambda b: (0, 0)),
            pl.BlockSpec((3, LANE), lambda b: (0, 0)),
            pl.BlockSpec((2, LANE), lambda b: (0, 0)),
        ],
        out_specs=pl.BlockSpec((1, 2, N), lambda b: (b, 0, 0)),
        out_shape=jax.ShapeDtypeStruct((B, 2, N), jnp.float32),
    )(Xf, M, brow, wab)

    return pl.pallas_call(
        _attn_kernel,
        grid=(N_IBLK, B),
        in_specs=[
            pl.BlockSpec((1, 2, I_BLK), lambda i, b: (b, 0, i)),
            pl.BlockSpec((1, 2, N), lambda i, b: (b, 0, 0)),
            pl.BlockSpec((I_BLK, N), lambda i, b: (i, 0)),
            pl.BlockSpec((1, 1), lambda i, b: (0, 0)),
        ],
        out_specs=pl.BlockSpec((1, I_BLK, N), lambda i, b: (b, i, 0)),
        out_shape=jax.ShapeDtypeStruct((B, N, N), jnp.float32),
    )(s_all, s_all, A, bfc2)


# speedup vs baseline: 10.9496x; 1.0472x over previous
import numpy as np

import jax
import jax.numpy as jnp
from jax.experimental import pallas as pl

B, N, T, C_IN = 8, 1000, 12, 64
OUT_CH = 4
ATT_DIM = T - 2
HALF = ATT_DIM * OUT_CH
LANE = 128
I_BLK = 1000
N_IBLK = (N + I_BLK - 1) // I_BLK

_SEL = np.zeros((3, T, ATT_DIM), dtype=np.float32)
for _k in range(3):
    for _t in range(ATT_DIM):
        _SEL[_k, _t + _k, _t] = 1.0


def _score_kernel(x_ref, m_ref, brow_ref, wab_ref, s_ref):
    x = x_ref[0]
    y = jnp.dot(x, m_ref[...], preferred_element_type=jnp.float32)
    o1 = y[:, 0:LANE] + brow_ref[0, :][None, :]
    o2 = y[:, LANE:2 * LANE] + brow_ref[1, :][None, :]
    o3 = y[:, 2 * LANE:3 * LANE] + brow_ref[2, :][None, :]
    t = jax.nn.relu(o1 + jax.nn.sigmoid(o2) + o3)
    s_ref[0] = jax.lax.dot_general(
        wab_ref[...], t, (((1,), (1,)), ((), ())),
        preferred_element_type=jnp.float32)


def _attn_kernel(s1_ref, s2_ref, a_ref, bfc_ref, out_ref):
    raw = s1_ref[0, 0, :][:, None] + s2_ref[0, 1, :][None, :] + bfc_ref[0, 0]
    scores = jnp.where(raw >= 0, raw, 0.01 * raw)
    a = a_ref[...]
    val = jnp.where(a != 0, scores, 0.0)
    m = jnp.max(val, axis=1, keepdims=True)
    e = jnp.exp(val - m)
    s = jnp.sum(e, axis=1, keepdims=True)
    out_ref[0] = e * (1.0 / s) * a


@jax.jit
def kernel(X, A, W1, b1, W2, b2, W3, b3, Wfc, bfc):
    W = jnp.stack([jnp.transpose(w[:, :, 0, :], (2, 1, 0))
                   for w in (W1, W2, W3)])
    sel = jnp.asarray(_SEL)
    M0 = jnp.einsum('kpt,jkco->pcjto', sel, W)
    M1 = M0.reshape(T * C_IN, 3, HALF)
    M = jnp.pad(M1, ((0, 0), (0, 0), (0, LANE - HALF))).reshape(
        T * C_IN, 3 * LANE)
    brow = jnp.pad(
        jnp.tile(jnp.stack([b1, b2, b3])[:, None, :], (1, ATT_DIM, 1))
        .reshape(3, HALF), ((0, 0), (0, LANE - HALF)))
    wab = jnp.pad(Wfc.reshape(2, HALF), ((0, 0), (0, LANE - HALF)))
    bfc2 = bfc.reshape(1, 1)
    Xf = X.reshape(B, N, T * C_IN)

    s_all = pl.pallas_call(
        _score_kernel,
        grid=(B,),
        in_specs=[
            pl.BlockSpec((1, N, T * C_IN), lambda b: (b, 0, 0)),
            pl.BlockSpec((T * C_IN, 3 * LANE), lambda b: (0, 0)),
            pl.BlockSpec((3, LANE), lambda b: (0, 0)),
            pl.BlockSpec((2, LANE), lambda b: (0, 0)),
        ],
        out_specs=pl.BlockSpec((1, 2, N), lambda b: (b, 0, 0)),
        out_shape=jax.ShapeDtypeStruct((B, 2, N), jnp.float32),
    )(Xf, M, brow, wab)

    return pl.pallas_call(
        _attn_kernel,
        grid=(N_IBLK, B),
        in_specs=[
            pl.BlockSpec((1, 2, I_BLK), lambda i, b: (b, 0, i)),
            pl.BlockSpec((1, 2, N), lambda i, b: (b, 0, 0)),
            pl.BlockSpec((I_BLK, N), lambda i, b: (i, 0)),
            pl.BlockSpec((1, 1), lambda i, b: (0, 0)),
        ],
        out_specs=pl.BlockSpec((1, I_BLK, N), lambda i, b: (b, i, 0)),
        out_shape=jax.ShapeDtypeStruct((B, N, N), jnp.float32),
    )(s_all, s_all, A, bfc2)


# speedup vs baseline: 11.5580x; 1.0556x over previous
import numpy as np

import jax
import jax.numpy as jnp
from jax.experimental import pallas as pl

B, N, T, C_IN = 8, 1000, 12, 64
OUT_CH = 4
ATT_DIM = T - 2
HALF = ATT_DIM * OUT_CH
LANE = 128

_SEL = np.zeros((3, T, ATT_DIM), dtype=np.float32)
for _k in range(3):
    for _t in range(ATT_DIM):
        _SEL[_k, _t + _k, _t] = 1.0


def _fused_kernel(x_ref, m_ref, brow_ref, wab_ref, wabt_ref, bfc_ref,
                  a_ref, out_ref):
    x = x_ref[0]
    y = jnp.dot(x, m_ref[...], preferred_element_type=jnp.float32)
    o1 = y[:, 0:LANE] + brow_ref[0, :][None, :]
    o2 = y[:, LANE:2 * LANE] + brow_ref[1, :][None, :]
    o3 = y[:, 2 * LANE:3 * LANE] + brow_ref[2, :][None, :]
    t = jax.nn.relu(o1 + jax.nn.sigmoid(o2) + o3)
    s1c = jnp.dot(t, wabt_ref[...],
                  preferred_element_type=jnp.float32)[:, 0:1]
    s2r = jax.lax.dot_general(
        wab_ref[...], t, (((1,), (1,)), ((), ())),
        preferred_element_type=jnp.float32)[1:2, :]
    raw = s1c + s2r + bfc_ref[0, 0]
    scores = jnp.where(raw >= 0, raw, 0.01 * raw)
    a = a_ref[...]
    val = jnp.where(a != 0, scores, 0.0)
    m = jnp.max(val, axis=1, keepdims=True)
    e = jnp.exp(val - m)
    s = jnp.sum(e, axis=1, keepdims=True)
    out_ref[0] = e * (1.0 / s) * a


@jax.jit
def kernel(X, A, W1, b1, W2, b2, W3, b3, Wfc, bfc):
    W = jnp.stack([jnp.transpose(w[:, :, 0, :], (2, 1, 0))
                   for w in (W1, W2, W3)])
    sel = jnp.asarray(_SEL)
    M0 = jnp.einsum('kpt,jkco->pcjto', sel, W)
    M1 = M0.reshape(T * C_IN, 3, HALF)
    M = jnp.pad(M1, ((0, 0), (0, 0), (0, LANE - HALF))).reshape(
        T * C_IN, 3 * LANE)
    brow = jnp.pad(
        jnp.tile(jnp.stack([b1, b2, b3])[:, None, :], (1, ATT_DIM, 1))
        .reshape(3, HALF), ((0, 0), (0, LANE - HALF)))
    wab = jnp.pad(Wfc.reshape(2, HALF), ((0, 0), (0, LANE - HALF)))
    wabt = wab.T
    bfc2 = bfc.reshape(1, 1)
    Xf = X.reshape(B, N, T * C_IN)

    return pl.pallas_call(
        _fused_kernel,
        grid=(B,),
        in_specs=[
            pl.BlockSpec((1, N, T * C_IN), lambda b: (b, 0, 0)),
            pl.BlockSpec((T * C_IN, 3 * LANE), lambda b: (0, 0)),
            pl.BlockSpec((3, LANE), lambda b: (0, 0)),
            pl.BlockSpec((2, LANE), lambda b: (0, 0)),
            pl.BlockSpec((LANE, 2), lambda b: (0, 0)),
            pl.BlockSpec((1, 1), lambda b: (0, 0)),
            pl.BlockSpec((N, N), lambda b: (0, 0)),
        ],
        out_specs=pl.BlockSpec((1, N, N), lambda b: (b, 0, 0)),
        out_shape=jax.ShapeDtypeStruct((B, N, N), jnp.float32),
    )(Xf, M, brow, wab, wabt, bfc2, A)
